# Initial kernel scaffold; baseline (speedup 1.0000x reference)
import jax
import jax.numpy as jnp
from jax.experimental import pallas as pl


def kernel(states_0, states_1, states_2, tm_0, tm_1, tm_2, norm_0, norm_1, norm_2, cm_0, cm_1, cm_2):
    raise NotImplementedError("write your pallas kernel here")


# speedup vs baseline: 2.8688x; 2.8688x over previous
import functools

import jax
import jax.numpy as jnp
from jax import lax
from jax.experimental import pallas as pl

_VOCAB = (256, 128, 128)
_K = 16
_S = 32
_JOINT = 256 * 128 * 128


def _tables_body(tm0_ref, tm1_ref, tm2_ref, w0_ref, w1_ref, w2_ref,
                 p0_ref, p1_ref, p2_ref):
    for tm_ref, w_ref, p_ref in ((tm0_ref, w0_ref, p0_ref),
                                 (tm1_ref, w1_ref, p1_ref),
                                 (tm2_ref, w2_ref, p2_ref)):
        x = tm_ref[0]
        w = w_ref[0]
        vals = jnp.sum(x * w, axis=1)
        vals = jnp.abs(vals) + 1e-9
        p_ref[0, 0] = vals / jnp.sum(vals)


def _compute_tables(tm0, tm1, tm2, w0, w1, w2):
    out = pl.pallas_call(
        _tables_body,
        grid=(_K,),
        in_specs=[
            pl.BlockSpec((1, _VOCAB[0], _S * _S), lambda k: (k, 0, 0)),
            pl.BlockSpec((1, _VOCAB[1], _S * _S), lambda k: (k, 0, 0)),
            pl.BlockSpec((1, _VOCAB[2], _S * _S), lambda k: (k, 0, 0)),
            pl.BlockSpec((1, 1, _S * _S), lambda k: (k, 0, 0)),
            pl.BlockSpec((1, 1, _S * _S), lambda k: (k, 0, 0)),
            pl.BlockSpec((1, 1, _S * _S), lambda k: (k, 0, 0)),
        ],
        out_specs=[
            pl.BlockSpec((1, 1, _VOCAB[0]), lambda k: (k, 0, 0)),
            pl.BlockSpec((1, 1, _VOCAB[1]), lambda k: (k, 0, 0)),
            pl.BlockSpec((1, 1, _VOCAB[2]), lambda k: (k, 0, 0)),
        ],
        out_shape=[
            jax.ShapeDtypeStruct((_K, 1, _VOCAB[0]), jnp.float32),
            jax.ShapeDtypeStruct((_K, 1, _VOCAB[1]), jnp.float32),
            jax.ShapeDtypeStruct((_K, 1, _VOCAB[2]), jnp.float32),
        ],
    )(tm0, tm1, tm2, w0, w1, w2)
    return [o.reshape(_K, v) for o, v in zip(out, _VOCAB)]


def _product_body(p0t_ref, p1_ref, p2_ref, cm0_ref, cm1_ref, cm2_ref,
                  raw_ref, ps_ref):
    kiota_c = lax.broadcasted_iota(jnp.int32, (_K, 128), 0)
    kiota_b = lax.broadcasted_iota(jnp.int32, (128, _K), 1)
    cm0v = cm0_ref[...]
    acc = jnp.zeros((128,), jnp.float32)
    for a in range(8):
        oh1 = (cm1_ref[a][None, :] == kiota_c).astype(jnp.float32)
        t1 = lax.dot_general(p1_ref[...], oh1, (((0,), (0,)), ((), ())),
                             preferred_element_type=jnp.float32)
        oh2 = (cm2_ref[a][:, None] == kiota_b).astype(jnp.float32)
        t2 = lax.dot_general(oh2, p2_ref[...], (((1,), (0,)), ((), ())),
                             preferred_element_type=jnp.float32)
        t0 = jnp.zeros((128, 128), jnp.float32)
        for k in range(_K):
            t0 = t0 + jnp.where(cm0v == k, p0t_ref[a, k], 0.0)
        slab = t0 * t1 * t2
        raw_ref[a] = slab
        acc = acc + jnp.sum(slab, axis=0)
    ps_ref[0, 0] = acc


def _compute_product(p0t, p1, p2, cm0, cm1, cm2):
    return pl.pallas_call(
        _product_body,
        grid=(32,),
        in_specs=[
            pl.BlockSpec((8, _K), lambda g: (g, 0)),
            pl.BlockSpec((_K, 128), lambda g: (0, 0)),
            pl.BlockSpec((_K, 128), lambda g: (0, 0)),
            pl.BlockSpec((128, 128), lambda g: (0, 0)),
            pl.BlockSpec((8, 128), lambda g: (g, 0)),
            pl.BlockSpec((8, 128), lambda g: (g, 0)),
        ],
        out_specs=[
            pl.BlockSpec((8, 128, 128), lambda g: (g, 0, 0)),
            pl.BlockSpec((1, 1, 128), lambda g: (g, 0, 0)),
        ],
        out_shape=[
            jax.ShapeDtypeStruct((256, 128, 128), jnp.float32),
            jax.ShapeDtypeStruct((32, 1, 128), jnp.float32),
        ],
    )(p0t, p1, p2, cm0, cm1, cm2)


def _scale_body(raw_ref, ps_ref, out_ref):
    total = jnp.sum(ps_ref[...])
    out_ref[...] = jnp.where(total > 0, raw_ref[...] / total, 1.0 / _JOINT)


def _scale(raw, partials):
    return pl.pallas_call(
        _scale_body,
        grid=(32,),
        in_specs=[
            pl.BlockSpec((8, 128, 128), lambda g: (g, 0, 0)),
            pl.BlockSpec((32, 1, 128), lambda g: (0, 0, 0)),
        ],
        out_specs=pl.BlockSpec((8, 128, 128), lambda g: (g, 0, 0)),
        out_shape=jax.ShapeDtypeStruct((256, 128, 128), jnp.float32),
    )(raw, partials)


@jax.jit
def kernel(states_0, states_1, states_2, tm_0, tm_1, tm_2,
           norm_0, norm_1, norm_2, cm_0, cm_1, cm_2):
    states = (states_0, states_1, states_2)
    tms = (tm_0, tm_1, tm_2)
    ws = (jnp.ones((_K, _S), jnp.float32), norm_1,
          jnp.ones((_K, _S), jnp.float32))
    tm_flat = [tms[i].reshape(_K, _VOCAB[i], _S * _S) for i in range(3)]
    w_flat = [(states[i][None, :, None] * ws[i][:, None, :])
              .reshape(_K, 1, _S * _S) for i in range(3)]

    p0, p1, p2 = _compute_tables(*tm_flat, *w_flat)

    cm0 = cm_0.reshape(128, 128)
    cm1 = cm_1.reshape(256, 128)
    cm2 = cm_2.reshape(256, 128)
    raw, partials = _compute_product(p0.T, p1, p2, cm0, cm1, cm2)
    out = _scale(raw, partials)
    return out.reshape(-1)
